# baseline (speedup 1.0000x reference)
import functools

import jax
import jax.numpy as jnp
from jax import lax
from jax.experimental import pallas as pl
from jax.experimental.pallas import tpu as pltpu
from jax.experimental.pallas import tpu_sc as plsc

NC = 2
NS = 16
LANES = 16
CHUNK = 512


def _emb_body(per_w, n_chunks, D,
              ids_hbm, tids_hbm, tok_hbm, typ_hbm, out_hbm,
              idx_v, tidx_v, rows_v, trows_v, sem0, sem1):
    wid = lax.axis_index("s") * NC + lax.axis_index("c")
    w_base = wid * per_w

    def chunk_body(ci, carry):
        base = w_base + ci * CHUNK
        pltpu.sync_copy(ids_hbm.at[pl.ds(base, CHUNK)], idx_v)
        pltpu.sync_copy(tids_hbm.at[pl.ds(base, CHUNK)], tidx_v)
        cp0 = pltpu.async_copy(tok_hbm.at[idx_v], rows_v, sem0)
        cp1 = pltpu.async_copy(typ_hbm.at[tidx_v], trows_v, sem1)
        cp0.wait()
        cp1.wait()

        def add_body(t, c2):
            for d in range(D // LANES):
                sl = pl.ds(d * LANES, LANES)
                rows_v[t, sl] = rows_v[t, sl] + trows_v[t, sl]
            return c2

        lax.fori_loop(0, CHUNK, add_body, 0, unroll=2)
        pltpu.sync_copy(rows_v, out_hbm.at[pl.ds(base, CHUNK)])
        return carry

    lax.fori_loop(0, n_chunks, chunk_body, 0)


def kernel(input_ids, type_ids, token_table, type_table):
    B, S = input_ids.shape
    V, D = token_table.shape
    N = B * S
    NW = NC * NS
    per_w = N // NW
    n_chunks = per_w // CHUNK
    assert per_w * NW == N and n_chunks * CHUNK == per_w

    ids = input_ids.reshape(N).astype(jnp.int32)
    tids = type_ids.reshape(N).astype(jnp.int32)

    mesh = plsc.VectorSubcoreMesh(
        core_axis_name="c", subcore_axis_name="s",
        num_cores=NC, num_subcores=NS)

    emb = functools.partial(
        pl.kernel,
        out_type=jax.ShapeDtypeStruct((N, D), jnp.float32),
        mesh=mesh,
        scratch_types=[
            pltpu.VMEM((CHUNK,), jnp.int32),
            pltpu.VMEM((CHUNK,), jnp.int32),
            pltpu.VMEM((CHUNK, D), jnp.float32),
            pltpu.VMEM((CHUNK, D), jnp.float32),
            pltpu.SemaphoreType.DMA,
            pltpu.SemaphoreType.DMA,
        ],
        compiler_params=pltpu.CompilerParams(use_tc_tiling_on_sc=False),
    )(functools.partial(_emb_body, per_w, n_chunks, D))

    out = emb(ids, tids, token_table, type_table)
    return out.reshape(B, S, D)
